# baseline (speedup 1.0000x reference)
import functools

import jax
import jax.numpy as jnp
from jax import lax
from jax.experimental import pallas as pl
from jax.experimental.pallas import tpu as pltpu
from jax.experimental.pallas import tpu_sc as plsc

_NC = 2
_NS = 16
_NW = _NC * _NS
_CH = 128
_NBUF = 4


@functools.lru_cache(maxsize=None)
def _build(B, V, D):
    assert B % (_NW * _CH) == 0, (B, _NW, _CH)
    bpw = B // _NW
    nch = bpw // _CH
    ngrp = nch // _NBUF
    assert nch % _NBUF == 0, (nch, _NBUF)

    mesh = plsc.VectorSubcoreMesh(core_axis_name="c", subcore_axis_name="s")

    @functools.partial(
        pl.kernel,
        mesh=mesh,
        out_type=jax.ShapeDtypeStruct((B, D), jnp.float32),
        scratch_types=[
            pltpu.VMEM((bpw,), jnp.int32),
            pltpu.VMEM((_NBUF, _CH, D), jnp.float32),
        ]
        + [pltpu.SemaphoreType.DMA] * (2 * _NBUF),
        compiler_params=pltpu.CompilerParams(use_tc_tiling_on_sc=False),
    )
    def gather_kernel(x_hbm, idx_hbm, out_hbm, idx_v, rows_v, *sems):
        gsems = sems[:_NBUF]
        wsems = sems[_NBUF:]
        wid = lax.axis_index("s") * _NC + lax.axis_index("c")
        base = wid * bpw

        pltpu.sync_copy(idx_hbm.at[pl.ds(base, bpw)], idx_v)

        def start_gather(j, b):
            pltpu.make_async_copy(
                x_hbm.at[idx_v.at[pl.ds(j * _CH, _CH)]],
                rows_v.at[b],
                gsems[b],
            ).start()

        def wait_gather(b):
            pltpu.make_async_copy(
                x_hbm.at[pl.ds(0, _CH)], rows_v.at[b], gsems[b]
            ).wait()

        def start_write(j, b):
            pltpu.make_async_copy(
                rows_v.at[b],
                out_hbm.at[pl.ds(base + j * _CH, _CH)],
                wsems[b],
            ).start()

        def wait_write(b):
            pltpu.make_async_copy(
                rows_v.at[b], out_hbm.at[pl.ds(0, _CH)], wsems[b]
            ).wait()

        for b in range(_NBUF):
            start_gather(b, b)

        def group(g, carry):
            for b in range(_NBUF):
                wait_gather(b)
                start_write(g * _NBUF + b, b)

            @pl.when(g + 1 < ngrp)
            def _():
                for b in range(_NBUF):
                    wait_write(b)
                    start_gather((g + 1) * _NBUF + b, b)

            return carry

        lax.fori_loop(0, ngrp, group, 0)

        for b in range(_NBUF):
            wait_write(b)

    return gather_kernel


def kernel(x, indices):
    V, D = x.shape
    (B,) = indices.shape
    idx = indices.astype(jnp.int32)
    return _build(B, V, D)(x, idx)


# speedup vs baseline: 1.0036x; 1.0036x over previous
import functools

import jax
import jax.numpy as jnp
from jax import lax
from jax.experimental import pallas as pl
from jax.experimental.pallas import tpu as pltpu
from jax.experimental.pallas import tpu_sc as plsc

_NC = 2
_NS = 16
_NW = _NC * _NS
_CH = 128
_NBUF = 4


@functools.lru_cache(maxsize=None)
def _build(B, V, D):
    assert B % (_NW * _CH) == 0, (B, _NW, _CH)
    bpw = B // _NW
    nch = bpw // _CH
    ngrp = nch // _NBUF
    assert nch % _NBUF == 0, (nch, _NBUF)

    mesh = plsc.VectorSubcoreMesh(core_axis_name="c", subcore_axis_name="s")

    @functools.partial(
        pl.kernel,
        mesh=mesh,
        out_type=jax.ShapeDtypeStruct((B, D), jnp.float32),
        scratch_types=[
            pltpu.VMEM((bpw,), jnp.int32),
            pltpu.VMEM((_NBUF, _CH, D), jnp.float32),
        ]
        + [pltpu.SemaphoreType.DMA] * (2 * _NBUF),
        compiler_params=pltpu.CompilerParams(
            use_tc_tiling_on_sc=False, skip_device_barrier=True
        ),
    )
    def gather_kernel(x_hbm, idx_hbm, out_hbm, idx_v, rows_v, *sems):
        gsems = sems[:_NBUF]
        wsems = sems[_NBUF:]
        wid = lax.axis_index("s") * _NC + lax.axis_index("c")
        base = wid * bpw

        pltpu.sync_copy(idx_hbm.at[pl.ds(base, bpw)], idx_v)

        def start_gather(j, b):
            pltpu.make_async_copy(
                x_hbm.at[idx_v.at[pl.ds(j * _CH, _CH)]],
                rows_v.at[b],
                gsems[b],
            ).start()

        def wait_gather(b):
            pltpu.make_async_copy(
                x_hbm.at[pl.ds(0, _CH)], rows_v.at[b], gsems[b]
            ).wait()

        def start_write(j, b):
            pltpu.make_async_copy(
                rows_v.at[b],
                out_hbm.at[pl.ds(base + j * _CH, _CH)],
                wsems[b],
            ).start()

        def wait_write(b):
            pltpu.make_async_copy(
                rows_v.at[b], out_hbm.at[pl.ds(0, _CH)], wsems[b]
            ).wait()

        for b in range(_NBUF):
            start_gather(b, b)

        def group(g, carry):
            for b in range(_NBUF):
                wait_gather(b)
                start_write(g * _NBUF + b, b)

            @pl.when(g + 1 < ngrp)
            def _():
                for b in range(_NBUF):
                    wait_write(b)
                    start_gather((g + 1) * _NBUF + b, b)

            return carry

        lax.fori_loop(0, ngrp, group, 0)

        for b in range(_NBUF):
            wait_write(b)

    return gather_kernel


def kernel(x, indices):
    V, D = x.shape
    (B,) = indices.shape
    idx = indices.astype(jnp.int32)
    return _build(B, V, D)(x, idx)
